# baseline (speedup 1.0000x reference)
import functools

import jax
import jax.numpy as jnp
from jax import lax
from jax.experimental import pallas as pl
from jax.experimental.pallas import tpu as pltpu
from jax.experimental.pallas import tpu_sc as plsc

N = 10000
E = 320000
F_IN = 128
H = 128
C = 40
CP = 48

NC, NS, L = 2, 16, 16
NW = NC * NS
K = 32
NBUF = 4
CHT = 320
JL = CHT // NBUF
EPT = CHT * K
EPAD = NW * EPT
NP = 10240
RPT = NP // NS


@functools.lru_cache(maxsize=None)
def _make_spmm(F):
  mesh = plsc.VectorSubcoreMesh(
      core_axis_name="c", subcore_axis_name="s",
      num_cores=NC, num_subcores=NS)

  @functools.partial(
      pl.kernel,
      out_type=jax.ShapeDtypeStruct((NC, NP, F), jnp.float32),
      mesh=mesh,
      scratch_types=[
          pltpu.VMEM((CHT, 2, K), jnp.int32),
          pltpu.VMEM((CHT, K), jnp.float32),
          pltpu.VMEM((K, F), jnp.float32),
          pltpu.VMEM((K, F), jnp.float32),
          pltpu.VMEM((K, F), jnp.float32),
          pltpu.VMEM((K, F), jnp.float32),
          pltpu.VMEM_SHARED((NP, F), jnp.float32),
          pltpu.SemaphoreType.DMA,
          pltpu.SemaphoreType.DMA,
          pltpu.SemaphoreType.DMA,
          pltpu.SemaphoreType.DMA,
      ],
      compiler_params=pltpu.CompilerParams(use_tc_tiling_on_sc=False),
  )
  def spmm(comb_hbm, w_hbm, sup_hbm, zero_hbm, out_hbm,
           comb, wbuf, rb0, rb1, rb2, rb3, acc,
           gs0, gs1, gs2, gs3):
    rbufs = (rb0, rb1, rb2, rb3)
    gsem = (gs0, gs1, gs2, gs3)
    cid = lax.axis_index("c")
    s = lax.axis_index("s")
    wid = s * NC + cid

    with jax.named_scope("stage"):
      pltpu.sync_copy(comb_hbm.at[wid], comb)
      pltpu.sync_copy(w_hbm.at[wid], wbuf)
      pltpu.sync_copy(zero_hbm.at[pl.ds(s * RPT, RPT)],
                      acc.at[pl.ds(s * RPT, RPT)])
      plsc.subcore_barrier()

    def gather_issue(c, b):
      pltpu.async_copy(sup_hbm.at[comb.at[c, 1]], rbufs[b], gsem[b])

    def gather_wait(c, b):
      pltpu.make_async_copy(sup_hbm.at[comb.at[c, 1]], rbufs[b],
                            gsem[b]).wait()

    def scatter_sync(c, b):
      pltpu.sync_copy(rbufs[b], acc.at[comb.at[c, 0]], add=True)

    with jax.named_scope("mainloop"):
      gather_issue(0, 0)
      gather_issue(1, 1)

      @pl.loop(0, JL)
      def _pipeline(j):
        for b in range(NBUF):
          c = j * NBUF + b
          rb = rbufs[b]
          gather_wait(c, b)

          @pl.loop(0, K // L)
          def _scale(t):
            wv = wbuf[c, pl.ds(t * L, L)]
            for i in range(L):
              e = t * L + i
              w = wv[i]
              for f in range(F // L):
                sl = pl.ds(f * L, L)
                rb[e, sl] = rb[e, sl] * w

          bo = (b + 2) % NBUF

          @pl.when(c < CHT - 2)
          def _():
            gather_issue(c + 2, bo)

          scatter_sync(c, b)

    with jax.named_scope("drain"):
      plsc.subcore_barrier()
      pltpu.sync_copy(acc.at[pl.ds(s * RPT, RPT)],
                      out_hbm.at[cid, pl.ds(s * RPT, RPT)])

  return spmm


def _tc1_body(x_ref, w_ref, o_ref):
  o_ref[...] = jnp.dot(x_ref[...], w_ref[...],
                       preferred_element_type=jnp.float32)


def _tc2_body(p_ref, b1_ref, w2_ref, o_ref):
  h = jnp.maximum(p_ref[0, :N] + p_ref[1, :N] + b1_ref[...], 0.0)
  o_ref[...] = jnp.dot(h, w2_ref[...], preferred_element_type=jnp.float32)


def _tc3_body(q_ref, b2_ref, o_ref):
  logits = q_ref[0, :N, :C] + q_ref[1, :N, :C] + b2_ref[...]
  m = jnp.max(logits, axis=1, keepdims=True)
  ex = jnp.exp(logits - m)
  lse = jnp.log(jnp.sum(ex, axis=1, keepdims=True))
  o_ref[...] = logits - m - lse


_tc1 = pl.pallas_call(
    _tc1_body, out_shape=jax.ShapeDtypeStruct((N, H), jnp.float32))
_tc2 = pl.pallas_call(
    _tc2_body, out_shape=jax.ShapeDtypeStruct((N, CP), jnp.float32))
_tc3 = pl.pallas_call(
    _tc3_body, out_shape=jax.ShapeDtypeStruct((N, C), jnp.float32))


def kernel(x, edge_index, edge_weight, W1, b1, W2, b2):
  pad = EPAD - E
  row = jnp.pad(edge_index[0], (0, pad)).reshape(NW, CHT, K)
  col = jnp.pad(edge_index[1], (0, pad)).reshape(NW, CHT, K)
  ew = jnp.pad(edge_weight, (0, pad)).reshape(NW, CHT, K)
  comb = jnp.stack([row, col], axis=2)
  w2p = jnp.pad(W2, ((0, 0), (0, CP - C)))

  support = _tc1(x, W1)
  part1 = _make_spmm(H)(comb, ew, support, jnp.zeros((NP, H), jnp.float32))
  support2 = _tc2(part1, b1, w2p)
  part2 = _make_spmm(CP)(comb, ew, support2,
                         jnp.zeros((NP, CP), jnp.float32))
  return _tc3(part2, b2)


# speedup vs baseline: 1.8045x; 1.8045x over previous
import functools

import jax
import jax.numpy as jnp
from jax import lax
from jax.experimental import pallas as pl
from jax.experimental.pallas import tpu as pltpu
from jax.experimental.pallas import tpu_sc as plsc

N = 10000
E = 320000
F_IN = 128
H = 128
C = 40
CP = 48

NC, NS, L = 2, 16, 16
NW = NC * NS
K = 32
NBUF = 4
CHT = 320
JL = CHT // NBUF
EPT = CHT * K
EPAD = NW * EPT
NP = 10240
RPT = NP // NS


@functools.lru_cache(maxsize=None)
def _make_spmm(F):
  mesh = plsc.VectorSubcoreMesh(
      core_axis_name="c", subcore_axis_name="s",
      num_cores=NC, num_subcores=NS)

  @functools.partial(
      pl.kernel,
      out_type=jax.ShapeDtypeStruct((NC, NP, F), jnp.float32),
      mesh=mesh,
      scratch_types=[
          pltpu.VMEM((CHT, 2, K), jnp.int32),
          pltpu.VMEM((CHT, K), jnp.float32),
          pltpu.VMEM((K, F), jnp.float32),
          pltpu.VMEM((K, F), jnp.float32),
          pltpu.VMEM((K, F), jnp.float32),
          pltpu.VMEM((K, F), jnp.float32),
          pltpu.VMEM_SHARED((NP, F), jnp.float32),
          pltpu.SemaphoreType.DMA,
          pltpu.SemaphoreType.DMA,
          pltpu.SemaphoreType.DMA,
          pltpu.SemaphoreType.DMA,
      ],
      compiler_params=pltpu.CompilerParams(use_tc_tiling_on_sc=False),
  )
  def spmm(comb_hbm, w_hbm, sup_hbm, zero_hbm, out_hbm,
           comb, wbuf, rb0, rb1, rb2, rb3, acc,
           gs0, gs1, gs2, gs3):
    rbufs = (rb0, rb1, rb2, rb3)
    gsem = (gs0, gs1, gs2, gs3)
    cid = lax.axis_index("c")
    s = lax.axis_index("s")
    wid = s * NC + cid

    with jax.named_scope("stage"):
      pltpu.sync_copy(comb_hbm.at[wid], comb)
      pltpu.sync_copy(w_hbm.at[wid], wbuf)
      pltpu.sync_copy(zero_hbm.at[pl.ds(s * RPT, RPT)],
                      acc.at[pl.ds(s * RPT, RPT)])
      plsc.subcore_barrier()

    def gather_issue(c, b):
      pltpu.async_copy(sup_hbm.at[comb.at[c, 1]], rbufs[b], gsem[b])

    def gather_wait(c, b):
      pltpu.make_async_copy(sup_hbm.at[comb.at[c, 1]], rbufs[b],
                            gsem[b]).wait()

    def scatter_sync(c, b):
      pltpu.sync_copy(rbufs[b], acc.at[comb.at[c, 0]], add=True)

    with jax.named_scope("mainloop"):
      gather_issue(0, 0)
      gather_issue(1, 1)

      @pl.loop(0, JL)
      def _pipeline(j):
        for b in range(NBUF):
          c = j * NBUF + b
          rb = rbufs[b]
          gather_wait(c, b)

          @pl.loop(0, K // L)
          def _scale(t):
            wv = wbuf[c, pl.ds(t * L, L)]
            for i in range(L):
              e = t * L + i
              w = wv[i]
              for f in range(F // L):
                sl = pl.ds(f * L, L)
                rb[e, sl] = rb[e, sl] * w

          bo = (b + 2) % NBUF

          @pl.when(c < CHT - 2)
          def _():
            gather_issue(c + 2, bo)

          scatter_sync(c, b)

    with jax.named_scope("drain"):
      plsc.subcore_barrier()
      pltpu.sync_copy(acc.at[pl.ds(s * RPT, RPT)],
                      out_hbm.at[cid, pl.ds(s * RPT, RPT)])

  return spmm


def _tc1_body(x_ref, w_ref, o_ref):
  o_ref[...] = jnp.dot(x_ref[...], w_ref[...],
                       preferred_element_type=jnp.float32)


def _tc2_body(p_ref, b1_ref, w2_ref, o_ref):
  h = jnp.maximum(p_ref[0, :N] + p_ref[1, :N] + b1_ref[...], 0.0)
  o_ref[...] = jnp.dot(h, w2_ref[...], preferred_element_type=jnp.float32)


def _tc3_body(q_ref, b2_ref, o_ref):
  logits = q_ref[0, :N, :C] + q_ref[1, :N, :C] + b2_ref[...]
  m = jnp.max(logits, axis=1, keepdims=True)
  ex = jnp.exp(logits - m)
  lse = jnp.log(jnp.sum(ex, axis=1, keepdims=True))
  o_ref[...] = logits - m - lse


_tc1 = pl.pallas_call(
    _tc1_body, out_shape=jax.ShapeDtypeStruct((N, H), jnp.float32))
_tc2 = pl.pallas_call(
    _tc2_body, out_shape=jax.ShapeDtypeStruct((N, CP), jnp.float32))
_tc3 = pl.pallas_call(
    _tc3_body, out_shape=jax.ShapeDtypeStruct((N, C), jnp.float32))


def kernel(x, edge_index, edge_weight, W1, b1, W2, b2):
  pad = EPAD - E
  pad_idx = jnp.arange(pad, dtype=jnp.int32)
  row = jnp.concatenate(
      [edge_index[0], N + pad_idx % (NP - N)]).reshape(NW, CHT, K)
  col = jnp.concatenate([edge_index[1], pad_idx % N]).reshape(NW, CHT, K)
  ew = jnp.pad(edge_weight, (0, pad)).reshape(NW, CHT, K)
  comb = jnp.stack([row, col], axis=2)
  w2p = jnp.pad(W2, ((0, 0), (0, CP - C)))

  support = _tc1(x, W1)
  part1 = _make_spmm(H)(comb, ew, support, jnp.zeros((NP, H), jnp.float32))
  support2 = _tc2(part1, b1, w2p)
  part2 = _make_spmm(CP)(comb, ew, support2,
                         jnp.zeros((NP, CP), jnp.float32))
  return _tc3(part2, b2)
